# baseline (speedup 1.0000x reference)
import functools

import jax
import jax.numpy as jnp
from jax.experimental import pallas as pl
from jax.experimental.pallas import tpu as pltpu

B = 16
POSE_DIM = 72
SD_DIM = 72
FC_DIM = 1024
LATENT_DIM = 256
NUM_EMB = 1024
COMMIT = 0.25


def _conv2d(x, w, b, stride, pad):
    y = jax.lax.conv_general_dilated(
        x.astype(jnp.bfloat16), w.transpose(2, 3, 1, 0).astype(jnp.bfloat16),
        (stride, stride), [(pad, pad), (pad, pad)],
        dimension_numbers=("NHWC", "HWIO", "NHWC"),
        preferred_element_type=jnp.float32)
    return y + b[None, None, None, :]


def _maxpool3x3s2(x):
    return jax.lax.reduce_window(x, -jnp.inf, jax.lax.max, (1, 3, 3, 1), (1, 2, 2, 1), "VALID")


def _c1_kernel(xs_ref, w2_ref, b_ref, out_ref, p2_ref):
    p2_ref[:, 55:64, :] = jnp.zeros((55, 9, 432), jnp.float32)
    for qy in range(3):
        for qx in range(3):
            q = qy * 3 + qx
            p2_ref[:, 0:55, 48 * q:48 * (q + 1)] = xs_ref[0, qy:qy + 55, qx:qx + 55, :]
    p2 = p2_ref[...].reshape(55 * 64, 432)
    acc = jax.lax.dot_general(
        p2.astype(jnp.bfloat16), w2_ref[...].astype(jnp.bfloat16),
        (((1,), (0,)), ((), ())), preferred_element_type=jnp.float32)
    acc = jax.nn.relu(acc + b_ref[...])
    out_ref[...] = acc.reshape(1, 55, 64, 64)


def _conv1_s2d_pallas(x, w, b):
    n = x.shape[0]
    xp = jnp.pad(x, ((0, 0), (0, 0), (2, 2), (2, 2)))
    xs = xp.reshape(n, 3, 57, 4, 57, 4)
    xs = xs.transpose(0, 2, 4, 1, 3, 5).reshape(n, 57, 57, 48)
    wp = jnp.pad(w, ((0, 0), (0, 0), (0, 1), (0, 1)))
    ws = wp.reshape(64, 3, 3, 4, 3, 4)
    ws = ws.transpose(0, 1, 3, 5, 2, 4).reshape(64, 48, 3, 3)
    w2 = ws.transpose(2, 3, 1, 0).reshape(432, 64)
    y = pl.pallas_call(
        _c1_kernel,
        grid=(n,),
        in_specs=[
            pl.BlockSpec((1, 57, 57, 48), lambda i: (i, 0, 0, 0)),
            pl.BlockSpec((432, 64), lambda i: (0, 0)),
            pl.BlockSpec((1, 64), lambda i: (0, 0)),
        ],
        out_specs=pl.BlockSpec((1, 55, 64, 64), lambda i: (i, 0, 0, 0)),
        out_shape=jax.ShapeDtypeStruct((n, 55, 64, 64), jnp.float32),
        scratch_shapes=[pltpu.VMEM((55, 64, 432), jnp.float32)],
    )(xs, w2, b.reshape(1, 64))
    return y[:, :, :55, :]


def _alexnet_features(x, p):
    x = _conv1_s2d_pallas(x, p["c1w"], p["c1b"])
    x = _maxpool3x3s2(x)
    x = jax.nn.relu(_conv2d(x, p["c2w"], p["c2b"], 1, 2))
    x = _maxpool3x3s2(x)
    x = jax.nn.relu(_conv2d(x, p["c3w"], p["c3b"], 1, 1))
    x = jax.nn.relu(_conv2d(x, p["c4w"], p["c4b"], 1, 1))
    x = jax.nn.relu(_conv2d(x, p["c5w"], p["c5b"], 1, 1))
    x = _maxpool3x3s2(x)
    x = x.transpose(0, 3, 1, 2)
    return x.reshape(x.shape[0], -1)


def _vq_kernel(lat_ref, emb_ref, loss_ref, q_ref, perp_ref):
    lat = lat_ref[...]
    emb = emb_ref[...]
    d = (jnp.sum(lat * lat, axis=1, keepdims=True)
         + jnp.sum(emb * emb, axis=1)[None, :]
         - 2.0 * jax.lax.dot_general(
             lat, emb, (((1,), (1,)), ((), ())),
             preferred_element_type=jnp.float32))
    idx = jnp.argmin(d, axis=1)
    enc = (jax.lax.broadcasted_iota(jnp.int32, (B, NUM_EMB), 1)
           == idx[:, None]).astype(jnp.float32)
    q = jax.lax.dot_general(enc, emb, (((1,), (0,)), ((), ())),
                            preferred_element_type=jnp.float32)
    q_ref[...] = q
    diff = q - lat
    loss_ref[...] = (COMMIT * jnp.mean(diff * diff)).reshape(1, 1)
    avg = jnp.mean(enc, axis=0)
    perp_ref[...] = jnp.exp(-jnp.sum(avg * jnp.log(avg + 1e-10))).reshape(1, 1)


@functools.partial(jax.jit, static_argnames=())
def _vq(latent, emb):
    loss, q, perp = pl.pallas_call(
        _vq_kernel,
        out_shape=(
            jax.ShapeDtypeStruct((1, 1), jnp.float32),
            jax.ShapeDtypeStruct((B, LATENT_DIM), jnp.float32),
            jax.ShapeDtypeStruct((1, 1), jnp.float32),
        ),
    )(latent, emb)
    return loss[0, 0], q, perp[0, 0]


def _condition_encoder(pose, img, img_crop, img_zoom, p):
    pf = jax.nn.relu(pose @ p["ce_fc1w"].T + p["ce_fc1b"])
    f = jnp.concatenate([_alexnet_features(img, p),
                         _alexnet_features(img_crop, p),
                         _alexnet_features(img_zoom, p)], axis=0)
    f = jax.nn.relu(f @ p["fc6w"].T + p["fc6b"])
    f = jax.nn.relu(f @ p["fc7w"].T + p["fc7b"])
    f1, f2, f3 = jnp.split(f, 3, axis=0)
    h = jnp.concatenate([pf, f1, f2, f3], axis=1)
    return jax.nn.relu(h @ p["ce_fc2w"].T + p["ce_fc2b"])


def kernel(x, pose, img, img_crop, img_zoom, params):
    p = params
    h = jax.nn.relu(x @ p["e_fc1w"].T + p["e_fc1b"])
    h = jax.nn.relu(h @ p["e_fc2w"].T + p["e_fc2b"])
    c = _condition_encoder(pose, img, img_crop, img_zoom, p)
    latent = jnp.concatenate([h, c], axis=1) @ p["e_flw"].T + p["e_flb"]
    loss, q, perp = _vq(latent, p["emb"])
    d = jax.nn.relu(q @ p["d_fc1w"].T + p["d_fc1b"])
    d = jax.nn.relu(d @ p["d_fc2w"].T + p["d_fc2b"])
    c2 = jax.nn.relu(c @ p["d_fc3w"].T + p["d_fc3b"])
    d = jnp.concatenate([d, c2], axis=1)
    d = jax.nn.relu(d @ p["d_fc4w"].T + p["d_fc4b"])
    d = jax.nn.relu(d @ p["d_fc5w"].T + p["d_fc5b"])
    x_recon = d @ p["d_fc6w"].T + p["d_fc6b"]
    return loss, x_recon, perp


# speedup vs baseline: 1.6168x; 1.6168x over previous
import functools

import jax
import jax.numpy as jnp
from jax.experimental import pallas as pl
from jax.experimental.pallas import tpu as pltpu
from jax.experimental.pallas import tpu_sc as plsc

B = 16
POSE_DIM = 72
SD_DIM = 72
FC_DIM = 1024
LATENT_DIM = 256
NUM_EMB = 1024
COMMIT = 0.25


def _conv2d(x, w, b, stride, pad):
    y = jax.lax.conv_general_dilated(
        x.astype(jnp.bfloat16), w.transpose(2, 3, 1, 0).astype(jnp.bfloat16),
        (stride, stride), [(pad, pad), (pad, pad)],
        dimension_numbers=("NHWC", "HWIO", "NHWC"),
        preferred_element_type=jnp.float32)
    return y + b[None, None, None, :]


def _maxpool3x3s2(x):
    return jax.lax.reduce_window(x, -jnp.inf, jax.lax.max, (1, 3, 3, 1), (1, 2, 2, 1), "VALID")


def _c1_kernel(xs_ref, w2_ref, b_ref, out_ref, p2_ref):
    p2_ref[:, 55:64, :] = jnp.zeros((55, 9, 432), jnp.float32)
    for qy in range(3):
        for qx in range(3):
            q = qy * 3 + qx
            p2_ref[:, 0:55, 48 * q:48 * (q + 1)] = xs_ref[0, qy:qy + 55, qx:qx + 55, :]
    p2 = p2_ref[...].reshape(55 * 64, 432)
    acc = jax.lax.dot_general(
        p2.astype(jnp.bfloat16), w2_ref[...].astype(jnp.bfloat16),
        (((1,), (0,)), ((), ())), preferred_element_type=jnp.float32)
    acc = jax.nn.relu(acc + b_ref[...])
    out_ref[...] = acc.reshape(1, 55, 64, 64)


def _conv1_s2d_pallas(x, w, b):
    n = x.shape[0]
    xp = jnp.pad(x, ((0, 0), (0, 0), (2, 2), (2, 2)))
    xs = xp.reshape(n, 3, 57, 4, 57, 4)
    xs = xs.transpose(0, 2, 4, 1, 3, 5).reshape(n, 57, 57, 48)
    wp = jnp.pad(w, ((0, 0), (0, 0), (0, 1), (0, 1)))
    ws = wp.reshape(64, 3, 3, 4, 3, 4)
    ws = ws.transpose(0, 1, 3, 5, 2, 4).reshape(64, 48, 3, 3)
    w2 = ws.transpose(2, 3, 1, 0).reshape(432, 64)
    y = pl.pallas_call(
        _c1_kernel,
        grid=(n,),
        in_specs=[
            pl.BlockSpec((1, 57, 57, 48), lambda i: (i, 0, 0, 0)),
            pl.BlockSpec((432, 64), lambda i: (0, 0)),
            pl.BlockSpec((1, 64), lambda i: (0, 0)),
        ],
        out_specs=pl.BlockSpec((1, 55, 64, 64), lambda i: (i, 0, 0, 0)),
        out_shape=jax.ShapeDtypeStruct((n, 55, 64, 64), jnp.float32),
        scratch_shapes=[pltpu.VMEM((55, 64, 432), jnp.float32)],
    )(xs, w2, b.reshape(1, 64))
    return y[:, :, :55, :]


def _alexnet_features(x, p):
    x = x.transpose(0, 2, 3, 1)
    x = jax.nn.relu(_conv2d(x, p["c1w"], p["c1b"], 4, 2))
    x = _maxpool3x3s2(x)
    x = jax.nn.relu(_conv2d(x, p["c2w"], p["c2b"], 1, 2))
    x = _maxpool3x3s2(x)
    x = jax.nn.relu(_conv2d(x, p["c3w"], p["c3b"], 1, 1))
    x = jax.nn.relu(_conv2d(x, p["c4w"], p["c4b"], 1, 1))
    x = jax.nn.relu(_conv2d(x, p["c5w"], p["c5b"], 1, 1))
    x = _maxpool3x3s2(x)
    x = x.transpose(0, 3, 1, 2)
    return x.reshape(x.shape[0], -1)


def _sc_gather_kernel(emb_hbm, idx_hbm, out_hbm, idx_v, rows_v, sem):
    first = ((jax.lax.axis_index("c") == 0) & (jax.lax.axis_index("s") == 0))

    @pl.when(first)
    def _():
        pltpu.sync_copy(idx_hbm, idx_v)
        pltpu.async_copy(emb_hbm.at[idx_v], rows_v, sem).wait()
        pltpu.sync_copy(rows_v, out_hbm)


def _sc_codebook_gather(emb, idx):
    kfn = functools.partial(
        pl.kernel,
        mesh=plsc.VectorSubcoreMesh(core_axis_name="c", subcore_axis_name="s"),
        out_type=jax.ShapeDtypeStruct((B, LATENT_DIM), jnp.float32),
        scratch_types=[
            pltpu.VMEM((B,), jnp.int32),
            pltpu.VMEM((B, LATENT_DIM), jnp.float32),
            pltpu.SemaphoreType.DMA,
        ],
    )
    return kfn(_sc_gather_kernel)(emb, idx)


def _vq_kernel(lat_ref, emb_ref, loss_ref, q_ref, perp_ref, idx_ref):
    lat = lat_ref[...]
    emb = emb_ref[...]
    d = (jnp.sum(lat * lat, axis=1, keepdims=True)
         + jnp.sum(emb * emb, axis=1)[None, :]
         - 2.0 * jax.lax.dot_general(
             lat, emb, (((1,), (1,)), ((), ())),
             preferred_element_type=jnp.float32))
    idx = jnp.argmin(d, axis=1)
    enc = (jax.lax.broadcasted_iota(jnp.int32, (B, NUM_EMB), 1)
           == idx[:, None]).astype(jnp.float32)
    q = jax.lax.dot_general(enc, emb, (((1,), (0,)), ((), ())),
                            preferred_element_type=jnp.float32)
    q_ref[...] = q
    idx_ref[...] = idx.reshape(B, 1)
    diff = q - lat
    loss_ref[...] = (COMMIT * jnp.mean(diff * diff)).reshape(1, 1)
    avg = jnp.mean(enc, axis=0)
    perp_ref[...] = jnp.exp(-jnp.sum(avg * jnp.log(avg + 1e-10))).reshape(1, 1)


@functools.partial(jax.jit, static_argnames=())
def _vq(latent, emb):
    loss, _q_tc, perp, idx = pl.pallas_call(
        _vq_kernel,
        out_shape=(
            jax.ShapeDtypeStruct((1, 1), jnp.float32),
            jax.ShapeDtypeStruct((B, LATENT_DIM), jnp.float32),
            jax.ShapeDtypeStruct((1, 1), jnp.float32),
            jax.ShapeDtypeStruct((B, 1), jnp.int32),
        ),
    )(latent, emb)
    q = _sc_codebook_gather(emb, idx.reshape(B))
    return loss[0, 0], q, perp[0, 0]


def _condition_encoder(pose, img, img_crop, img_zoom, p):
    pf = jax.nn.relu(pose @ p["ce_fc1w"].T + p["ce_fc1b"])
    f = jnp.concatenate([_alexnet_features(img, p),
                         _alexnet_features(img_crop, p),
                         _alexnet_features(img_zoom, p)], axis=0)
    f = jax.nn.relu(f @ p["fc6w"].T + p["fc6b"])
    f = jax.nn.relu(f @ p["fc7w"].T + p["fc7b"])
    f1, f2, f3 = jnp.split(f, 3, axis=0)
    h = jnp.concatenate([pf, f1, f2, f3], axis=1)
    return jax.nn.relu(h @ p["ce_fc2w"].T + p["ce_fc2b"])


def kernel(x, pose, img, img_crop, img_zoom, params):
    p = params
    h = jax.nn.relu(x @ p["e_fc1w"].T + p["e_fc1b"])
    h = jax.nn.relu(h @ p["e_fc2w"].T + p["e_fc2b"])
    c = _condition_encoder(pose, img, img_crop, img_zoom, p)
    latent = jnp.concatenate([h, c], axis=1) @ p["e_flw"].T + p["e_flb"]
    loss, q, perp = _vq(latent, p["emb"])
    d = jax.nn.relu(q @ p["d_fc1w"].T + p["d_fc1b"])
    d = jax.nn.relu(d @ p["d_fc2w"].T + p["d_fc2b"])
    c2 = jax.nn.relu(c @ p["d_fc3w"].T + p["d_fc3b"])
    d = jnp.concatenate([d, c2], axis=1)
    d = jax.nn.relu(d @ p["d_fc4w"].T + p["d_fc4b"])
    d = jax.nn.relu(d @ p["d_fc5w"].T + p["d_fc5b"])
    x_recon = d @ p["d_fc6w"].T + p["d_fc6b"]
    return loss, x_recon, perp


# speedup vs baseline: 1.6332x; 1.0102x over previous
import functools

import jax
import jax.numpy as jnp
from jax.experimental import pallas as pl
from jax.experimental.pallas import tpu as pltpu
from jax.experimental.pallas import tpu_sc as plsc

B = 16
POSE_DIM = 72
SD_DIM = 72
FC_DIM = 1024
LATENT_DIM = 256
NUM_EMB = 1024
COMMIT = 0.25


def _conv2d(x, w, b, stride, pad):
    y = jax.lax.conv_general_dilated(
        x.astype(jnp.bfloat16), w.transpose(2, 3, 1, 0).astype(jnp.bfloat16),
        (stride, stride), [(pad, pad), (pad, pad)],
        dimension_numbers=("NHWC", "HWIO", "NHWC"),
        preferred_element_type=jnp.float32)
    return y + b[None, None, None, :]


def _maxpool3x3s2(x):
    return jax.lax.reduce_window(x, -jnp.inf, jax.lax.max, (1, 3, 3, 1), (1, 2, 2, 1), "VALID")


def _c1_kernel(xs_ref, w2_ref, b_ref, out_ref, p2_ref):
    p2_ref[:, 55:64, :] = jnp.zeros((55, 9, 432), jnp.float32)
    for qy in range(3):
        for qx in range(3):
            q = qy * 3 + qx
            p2_ref[:, 0:55, 48 * q:48 * (q + 1)] = xs_ref[0, qy:qy + 55, qx:qx + 55, :]
    p2 = p2_ref[...].reshape(55 * 64, 432)
    acc = jax.lax.dot_general(
        p2.astype(jnp.bfloat16), w2_ref[...].astype(jnp.bfloat16),
        (((1,), (0,)), ((), ())), preferred_element_type=jnp.float32)
    acc = jax.nn.relu(acc + b_ref[...])
    out_ref[...] = acc.reshape(1, 55, 64, 64)


def _conv1_s2d_pallas(x, w, b):
    n = x.shape[0]
    xp = jnp.pad(x, ((0, 0), (0, 0), (2, 2), (2, 2)))
    xs = xp.reshape(n, 3, 57, 4, 57, 4)
    xs = xs.transpose(0, 2, 4, 1, 3, 5).reshape(n, 57, 57, 48)
    wp = jnp.pad(w, ((0, 0), (0, 0), (0, 1), (0, 1)))
    ws = wp.reshape(64, 3, 3, 4, 3, 4)
    ws = ws.transpose(0, 1, 3, 5, 2, 4).reshape(64, 48, 3, 3)
    w2 = ws.transpose(2, 3, 1, 0).reshape(432, 64)
    y = pl.pallas_call(
        _c1_kernel,
        grid=(n,),
        in_specs=[
            pl.BlockSpec((1, 57, 57, 48), lambda i: (i, 0, 0, 0)),
            pl.BlockSpec((432, 64), lambda i: (0, 0)),
            pl.BlockSpec((1, 64), lambda i: (0, 0)),
        ],
        out_specs=pl.BlockSpec((1, 55, 64, 64), lambda i: (i, 0, 0, 0)),
        out_shape=jax.ShapeDtypeStruct((n, 55, 64, 64), jnp.float32),
        scratch_shapes=[pltpu.VMEM((55, 64, 432), jnp.float32)],
    )(xs, w2, b.reshape(1, 64))
    return y[:, :, :55, :]


def _alexnet_features(x, p):
    x = x.transpose(0, 2, 3, 1)
    x = jax.nn.relu(_conv2d(x, p["c1w"], p["c1b"], 4, 2))
    x = _maxpool3x3s2(x)
    x = jax.nn.relu(_conv2d(x, p["c2w"], p["c2b"], 1, 2))
    x = _maxpool3x3s2(x)
    x = jax.nn.relu(_conv2d(x, p["c3w"], p["c3b"], 1, 1))
    x = jax.nn.relu(_conv2d(x, p["c4w"], p["c4b"], 1, 1))
    x = jax.nn.relu(_conv2d(x, p["c5w"], p["c5b"], 1, 1))
    x = _maxpool3x3s2(x)
    x = x.transpose(0, 3, 1, 2)
    return x.reshape(x.shape[0], -1)


def _sc_gather_kernel(emb_hbm, idx_hbm, out_hbm, idx_v, rows_v, sem):
    first = ((jax.lax.axis_index("c") == 0) & (jax.lax.axis_index("s") == 0))

    @pl.when(first)
    def _():
        pltpu.sync_copy(idx_hbm, idx_v)
        pltpu.async_copy(emb_hbm.at[idx_v], rows_v, sem).wait()
        pltpu.sync_copy(rows_v, out_hbm)


def _sc_codebook_gather(emb, idx):
    kfn = functools.partial(
        pl.kernel,
        mesh=plsc.VectorSubcoreMesh(core_axis_name="c", subcore_axis_name="s"),
        out_type=jax.ShapeDtypeStruct((B, LATENT_DIM), jnp.float32),
        scratch_types=[
            pltpu.VMEM((B,), jnp.int32),
            pltpu.VMEM((B, LATENT_DIM), jnp.float32),
            pltpu.SemaphoreType.DMA,
        ],
    )
    return kfn(_sc_gather_kernel)(emb, idx)


def _mm_t(a, w):
    return jax.lax.dot_general(a, w, (((1,), (1,)), ((), ())),
                               preferred_element_type=jnp.float32)


def _vq_kernel(x_ref, c_ref, e1w_ref, e1b_ref, e2w_ref, e2b_ref, flw_ref,
               flb_ref, emb_ref, loss_ref, q_ref, perp_ref, idx_ref):
    h = jax.nn.relu(_mm_t(x_ref[...], e1w_ref[...]) + e1b_ref[...])
    h = jax.nn.relu(_mm_t(h, e2w_ref[...]) + e2b_ref[...])
    flw = flw_ref[...]
    lat = (_mm_t(h, flw[:, :FC_DIM]) + _mm_t(c_ref[...], flw[:, FC_DIM:])
           + flb_ref[...])
    emb = emb_ref[...]
    d = (jnp.sum(lat * lat, axis=1, keepdims=True)
         + jnp.sum(emb * emb, axis=1)[None, :]
         - 2.0 * jax.lax.dot_general(
             lat, emb, (((1,), (1,)), ((), ())),
             preferred_element_type=jnp.float32))
    idx = jnp.argmin(d, axis=1)
    enc = (jax.lax.broadcasted_iota(jnp.int32, (B, NUM_EMB), 1)
           == idx[:, None]).astype(jnp.float32)
    q = jax.lax.dot_general(enc, emb, (((1,), (0,)), ((), ())),
                            preferred_element_type=jnp.float32)
    q_ref[...] = q
    idx_ref[...] = idx.reshape(B, 1)
    diff = q - lat
    loss_ref[...] = (COMMIT * jnp.mean(diff * diff)).reshape(1, 1)
    avg = jnp.mean(enc, axis=0)
    perp_ref[...] = jnp.exp(-jnp.sum(avg * jnp.log(avg + 1e-10))).reshape(1, 1)


def _enc_vq(x, c, p):
    loss, _q_tc, perp, idx = pl.pallas_call(
        _vq_kernel,
        out_shape=(
            jax.ShapeDtypeStruct((1, 1), jnp.float32),
            jax.ShapeDtypeStruct((B, LATENT_DIM), jnp.float32),
            jax.ShapeDtypeStruct((1, 1), jnp.float32),
            jax.ShapeDtypeStruct((B, 1), jnp.int32),
        ),
    )(x, c, p["e_fc1w"], p["e_fc1b"].reshape(1, -1), p["e_fc2w"],
      p["e_fc2b"].reshape(1, -1), p["e_flw"], p["e_flb"].reshape(1, -1),
      p["emb"])
    q = _sc_codebook_gather(p["emb"], idx.reshape(B))
    return loss[0, 0], q, perp[0, 0]


def _dec_kernel(q_ref, c_ref, d1w_ref, d1b_ref, d2w_ref, d2b_ref, d3w_ref,
                d3b_ref, d4w_ref, d4b_ref, d5w_ref, d5b_ref, d6w_ref,
                d6b_ref, out_ref):
    d = jax.nn.relu(_mm_t(q_ref[...], d1w_ref[...]) + d1b_ref[...])
    d = jax.nn.relu(_mm_t(d, d2w_ref[...]) + d2b_ref[...])
    c2 = jax.nn.relu(_mm_t(c_ref[...], d3w_ref[...]) + d3b_ref[...])
    d4w = d4w_ref[...]
    d = jax.nn.relu(_mm_t(d, d4w[:, :FC_DIM]) + _mm_t(c2, d4w[:, FC_DIM:])
                    + d4b_ref[...])
    d = jax.nn.relu(_mm_t(d, d5w_ref[...]) + d5b_ref[...])
    out_ref[...] = _mm_t(d, d6w_ref[...]) + d6b_ref[...]


def _decoder(q, c, p):
    return pl.pallas_call(
        _dec_kernel,
        out_shape=jax.ShapeDtypeStruct((B, SD_DIM), jnp.float32),
    )(q, c, p["d_fc1w"], p["d_fc1b"].reshape(1, -1), p["d_fc2w"],
      p["d_fc2b"].reshape(1, -1), p["d_fc3w"], p["d_fc3b"].reshape(1, -1),
      p["d_fc4w"], p["d_fc4b"].reshape(1, -1), p["d_fc5w"],
      p["d_fc5b"].reshape(1, -1), p["d_fc6w"], p["d_fc6b"].reshape(1, -1))


def _condition_encoder(pose, img, img_crop, img_zoom, p):
    pf = jax.nn.relu(pose @ p["ce_fc1w"].T + p["ce_fc1b"])
    f = jnp.concatenate([_alexnet_features(img, p),
                         _alexnet_features(img_crop, p),
                         _alexnet_features(img_zoom, p)], axis=0)
    f = jax.nn.relu(f @ p["fc6w"].T + p["fc6b"])
    f = jax.nn.relu(f @ p["fc7w"].T + p["fc7b"])
    f1, f2, f3 = jnp.split(f, 3, axis=0)
    h = jnp.concatenate([pf, f1, f2, f3], axis=1)
    return jax.nn.relu(h @ p["ce_fc2w"].T + p["ce_fc2b"])


def kernel(x, pose, img, img_crop, img_zoom, params):
    p = params
    c = _condition_encoder(pose, img, img_crop, img_zoom, p)
    loss, q, perp = _enc_vq(x, c, p)
    x_recon = _decoder(q, c, p)
    return loss, x_recon, perp


# speedup vs baseline: 1.6371x; 1.0024x over previous
import functools

import jax
import jax.numpy as jnp
from jax.experimental import pallas as pl
from jax.experimental.pallas import tpu as pltpu
from jax.experimental.pallas import tpu_sc as plsc

B = 16
POSE_DIM = 72
SD_DIM = 72
FC_DIM = 1024
LATENT_DIM = 256
NUM_EMB = 1024
COMMIT = 0.25


def _conv2d(x, w, b, stride, pad):
    y = jax.lax.conv_general_dilated(
        x.astype(jnp.bfloat16), w.transpose(2, 3, 1, 0).astype(jnp.bfloat16),
        (stride, stride), [(pad, pad), (pad, pad)],
        dimension_numbers=("NHWC", "HWIO", "NHWC"),
        preferred_element_type=jnp.float32)
    return y + b[None, None, None, :]


def _maxpool3x3s2(x):
    return jax.lax.reduce_window(x, -jnp.inf, jax.lax.max, (1, 3, 3, 1), (1, 2, 2, 1), "VALID")


def _c1_kernel(xs_ref, w2_ref, b_ref, out_ref, p2_ref):
    p2_ref[:, 55:64, :] = jnp.zeros((55, 9, 432), jnp.float32)
    for qy in range(3):
        for qx in range(3):
            q = qy * 3 + qx
            p2_ref[:, 0:55, 48 * q:48 * (q + 1)] = xs_ref[0, qy:qy + 55, qx:qx + 55, :]
    p2 = p2_ref[...].reshape(55 * 64, 432)
    acc = jax.lax.dot_general(
        p2.astype(jnp.bfloat16), w2_ref[...].astype(jnp.bfloat16),
        (((1,), (0,)), ((), ())), preferred_element_type=jnp.float32)
    acc = jax.nn.relu(acc + b_ref[...])
    out_ref[...] = acc.reshape(1, 55, 64, 64)


def _conv1_s2d_pallas(x, w, b):
    n = x.shape[0]
    xp = jnp.pad(x, ((0, 0), (0, 0), (2, 2), (2, 2)))
    xs = xp.reshape(n, 3, 57, 4, 57, 4)
    xs = xs.transpose(0, 2, 4, 1, 3, 5).reshape(n, 57, 57, 48)
    wp = jnp.pad(w, ((0, 0), (0, 0), (0, 1), (0, 1)))
    ws = wp.reshape(64, 3, 3, 4, 3, 4)
    ws = ws.transpose(0, 1, 3, 5, 2, 4).reshape(64, 48, 3, 3)
    w2 = ws.transpose(2, 3, 1, 0).reshape(432, 64)
    y = pl.pallas_call(
        _c1_kernel,
        grid=(n,),
        in_specs=[
            pl.BlockSpec((1, 57, 57, 48), lambda i: (i, 0, 0, 0)),
            pl.BlockSpec((432, 64), lambda i: (0, 0)),
            pl.BlockSpec((1, 64), lambda i: (0, 0)),
        ],
        out_specs=pl.BlockSpec((1, 55, 64, 64), lambda i: (i, 0, 0, 0)),
        out_shape=jax.ShapeDtypeStruct((n, 55, 64, 64), jnp.float32),
        scratch_shapes=[pltpu.VMEM((55, 64, 432), jnp.float32)],
    )(xs, w2, b.reshape(1, 64))
    return y[:, :, :55, :]


def _alexnet_features(x, p):
    x = x.transpose(0, 2, 3, 1)
    x = jax.nn.relu(_conv2d(x, p["c1w"], p["c1b"], 4, 2))
    x = _maxpool3x3s2(x)
    x = jax.nn.relu(_conv2d(x, p["c2w"], p["c2b"], 1, 2))
    x = _maxpool3x3s2(x)
    x = jax.nn.relu(_conv2d(x, p["c3w"], p["c3b"], 1, 1))
    x = jax.nn.relu(_conv2d(x, p["c4w"], p["c4b"], 1, 1))
    x = jax.nn.relu(_conv2d(x, p["c5w"], p["c5b"], 1, 1))
    x = _maxpool3x3s2(x)
    x = x.transpose(0, 3, 1, 2)
    return x.reshape(x.shape[0], -1)


def _sc_gather_kernel(emb_hbm, idx_hbm, out_hbm, idx_v, rows_v, sem):
    first = ((jax.lax.axis_index("c") == 0) & (jax.lax.axis_index("s") == 0))

    @pl.when(first)
    def _():
        pltpu.sync_copy(idx_hbm, idx_v)
        pltpu.async_copy(emb_hbm.at[idx_v], rows_v, sem).wait()
        pltpu.sync_copy(rows_v, out_hbm)


def _sc_codebook_gather(emb, idx):
    kfn = functools.partial(
        pl.kernel,
        mesh=plsc.VectorSubcoreMesh(core_axis_name="c", subcore_axis_name="s"),
        out_type=jax.ShapeDtypeStruct((B, LATENT_DIM), jnp.float32),
        scratch_types=[
            pltpu.VMEM((B,), jnp.int32),
            pltpu.VMEM((B, LATENT_DIM), jnp.float32),
            pltpu.SemaphoreType.DMA,
        ],
    )
    return kfn(_sc_gather_kernel)(emb, idx)


def _mm_t(a, w):
    return jax.lax.dot_general(a, w, (((1,), (1,)), ((), ())),
                               preferred_element_type=jnp.float32)


def _vq_kernel(x_ref, c_ref, e1w_ref, e1b_ref, e2w_ref, e2b_ref, flw_ref,
               flb_ref, emb_ref, loss_ref, q_ref, perp_ref, idx_ref):
    h = jax.nn.relu(_mm_t(x_ref[...], e1w_ref[...]) + e1b_ref[...])
    h = jax.nn.relu(_mm_t(h, e2w_ref[...]) + e2b_ref[...])
    hc = jnp.concatenate([h, c_ref[...]], axis=1)
    lat = _mm_t(hc, flw_ref[...]) + flb_ref[...]
    emb = emb_ref[...]
    d = (jnp.sum(lat * lat, axis=1, keepdims=True)
         + jnp.sum(emb * emb, axis=1)[None, :]
         - 2.0 * jax.lax.dot_general(
             lat, emb, (((1,), (1,)), ((), ())),
             preferred_element_type=jnp.float32))
    idx = jnp.argmin(d, axis=1)
    enc = (jax.lax.broadcasted_iota(jnp.int32, (B, NUM_EMB), 1)
           == idx[:, None]).astype(jnp.float32)
    q = jax.lax.dot_general(enc, emb, (((1,), (0,)), ((), ())),
                            preferred_element_type=jnp.float32)
    q_ref[...] = q
    idx_ref[...] = idx.reshape(B, 1)
    diff = q - lat
    loss_ref[...] = (COMMIT * jnp.mean(diff * diff)).reshape(1, 1)
    avg = jnp.mean(enc, axis=0)
    perp_ref[...] = jnp.exp(-jnp.sum(avg * jnp.log(avg + 1e-10))).reshape(1, 1)


def _enc_vq(x, c, p):
    loss, _q_tc, perp, idx = pl.pallas_call(
        _vq_kernel,
        out_shape=(
            jax.ShapeDtypeStruct((1, 1), jnp.float32),
            jax.ShapeDtypeStruct((B, LATENT_DIM), jnp.float32),
            jax.ShapeDtypeStruct((1, 1), jnp.float32),
            jax.ShapeDtypeStruct((B, 1), jnp.int32),
        ),
    )(x, c, p["e_fc1w"], p["e_fc1b"].reshape(1, -1), p["e_fc2w"],
      p["e_fc2b"].reshape(1, -1), p["e_flw"], p["e_flb"].reshape(1, -1),
      p["emb"])
    q = _sc_codebook_gather(p["emb"], idx.reshape(B))
    return loss[0, 0], q, perp[0, 0]


def _dec_kernel(q_ref, c_ref, d1w_ref, d1b_ref, d2w_ref, d2b_ref, d3w_ref,
                d3b_ref, d4w_ref, d4b_ref, d5w_ref, d5b_ref, d6w_ref,
                d6b_ref, out_ref):
    d = jax.nn.relu(_mm_t(q_ref[...], d1w_ref[...]) + d1b_ref[...])
    d = jax.nn.relu(_mm_t(d, d2w_ref[...]) + d2b_ref[...])
    c2 = jax.nn.relu(_mm_t(c_ref[...], d3w_ref[...]) + d3b_ref[...])
    dc = jnp.concatenate([d, c2], axis=1)
    d = jax.nn.relu(_mm_t(dc, d4w_ref[...]) + d4b_ref[...])
    d = jax.nn.relu(_mm_t(d, d5w_ref[...]) + d5b_ref[...])
    out_ref[...] = _mm_t(d, d6w_ref[...]) + d6b_ref[...]


def _decoder(q, c, p):
    return pl.pallas_call(
        _dec_kernel,
        out_shape=jax.ShapeDtypeStruct((B, SD_DIM), jnp.float32),
    )(q, c, p["d_fc1w"], p["d_fc1b"].reshape(1, -1), p["d_fc2w"],
      p["d_fc2b"].reshape(1, -1), p["d_fc3w"], p["d_fc3b"].reshape(1, -1),
      p["d_fc4w"], p["d_fc4b"].reshape(1, -1), p["d_fc5w"],
      p["d_fc5b"].reshape(1, -1), p["d_fc6w"], p["d_fc6b"].reshape(1, -1))


def _condition_encoder(pose, img, img_crop, img_zoom, p):
    pf = jax.nn.relu(pose @ p["ce_fc1w"].T + p["ce_fc1b"])
    f = jnp.concatenate([_alexnet_features(img, p),
                         _alexnet_features(img_crop, p),
                         _alexnet_features(img_zoom, p)], axis=0)
    f = jax.nn.relu(f @ p["fc6w"].T + p["fc6b"])
    f = jax.nn.relu(f @ p["fc7w"].T + p["fc7b"])
    f1, f2, f3 = jnp.split(f, 3, axis=0)
    h = jnp.concatenate([pf, f1, f2, f3], axis=1)
    return jax.nn.relu(h @ p["ce_fc2w"].T + p["ce_fc2b"])


def kernel(x, pose, img, img_crop, img_zoom, params):
    p = params
    c = _condition_encoder(pose, img, img_crop, img_zoom, p)
    loss, q, perp = _enc_vq(x, c, p)
    x_recon = _decoder(q, c, p)
    return loss, x_recon, perp
